# baseline (speedup 1.0000x reference)
import functools

import jax
import jax.numpy as jnp
from jax import lax
from jax.experimental import pallas as pl
from jax.experimental.pallas import tpu as pltpu
from jax.experimental.pallas import tpu_sc as plsc

J = 0.1
_W = 4096

_NC, _NS = 2, 16
_NW = _NC * _NS


def _sc_gather(flat_ref, idx_ref, out_ref, idx_v, val_v, sem, *, b_per_w):
    wid = lax.axis_index("s") * _NC + lax.axis_index("c")
    base = wid * b_per_w
    pltpu.sync_copy(idx_ref.at[pl.ds(base, b_per_w)], idx_v)
    pltpu.async_copy(flat_ref.at[idx_v], val_v, sem).wait()
    pltpu.sync_copy(val_v, out_ref.at[pl.ds(base, b_per_w)])


@functools.cache
def _make_gather(B):
    b_per_w = B // _NW
    mesh = plsc.VectorSubcoreMesh(
        core_axis_name="c", subcore_axis_name="s",
        num_cores=_NC, num_subcores=_NS,
    )
    return pl.kernel(
        functools.partial(_sc_gather, b_per_w=b_per_w),
        out_type=jax.ShapeDtypeStruct((B,), jnp.float32),
        mesh=mesh,
        scratch_types=[
            pltpu.VMEM((b_per_w,), jnp.int32),
            pltpu.VMEM((b_per_w,), jnp.float32),
            pltpu.SemaphoreType.DMA,
        ],
    )


def _tc_body(x_ref, cor_ref, out_ref, acc_ref, *, n_cols):
    k = pl.program_id(0)

    @pl.when(k == 0)
    def _init():
        acc_ref[0] = 0.0

    x = x_ref[...]
    ids = k * _W + lax.broadcasted_iota(jnp.int32, x.shape, 1)
    acc_ref[0] += jnp.sum(jnp.where(ids < n_cols, x, 0.0))

    @pl.when(k == pl.num_programs(0) - 1)
    def _fin():
        csum = jnp.sum(cor_ref[...])
        b = cor_ref.shape[0] * cor_ref.shape[1]
        val = (acc_ref[0] - (n_cols + 1.0) * csum) / b + (2.0 * n_cols - 1.0) * J
        out_ref[...] = jnp.reshape(val, (1, 1))


def kernel(outputs, labels):
    B, V = outputs.shape
    lab = labels.astype(jnp.int32)
    flat_idx = jnp.arange(B, dtype=jnp.int32) * V + lab
    correct = _make_gather(B)(outputs.reshape(B * V), flat_idx)

    n_blocks = pl.cdiv(V, _W)
    out = pl.pallas_call(
        functools.partial(_tc_body, n_cols=V),
        grid=(n_blocks,),
        in_specs=[
            pl.BlockSpec((B, _W), lambda k: (0, k)),
            pl.BlockSpec((8, B // 8), lambda k: (0, 0)),
        ],
        out_specs=pl.BlockSpec((1, 1), lambda k: (0, 0)),
        out_shape=jax.ShapeDtypeStruct((1, 1), jnp.float32),
        scratch_shapes=[pltpu.SMEM((1,), jnp.float32)],
    )(outputs, correct.reshape(8, B // 8))
    return out[0, 0]


# speedup vs baseline: 2.1295x; 2.1295x over previous
import functools

import jax
import jax.numpy as jnp
from jax import lax
from jax.experimental import pallas as pl
from jax.experimental.pallas import tpu as pltpu
from jax.experimental.pallas import tpu_sc as plsc

J = 0.1
_W = 4096

_NC, _NS = 2, 16
_NW = _NC * _NS
_L = 16


def _sc_gather(x_ref, lab_ref, out_ref, lab_v, win_v, acc_v, sem, *, b_per_w):
    wid = lax.axis_index("s") * _NC + lax.axis_index("c")
    base = wid * b_per_w
    pltpu.sync_copy(lab_ref.at[pl.ds(base, b_per_w)], lab_v)

    chunks = [lab_v[pl.ds(c * _L, _L)] for c in range(b_per_w // _L)]
    cols = [chunks[i // _L][i % _L] for i in range(b_per_w)]

    copies = []
    for i in range(b_per_w):
        a = (cols[i] // _L) * _L
        copies.append(
            pltpu.async_copy(x_ref.at[base + i, pl.ds(a, _L)], win_v.at[i], sem)
        )
    for c in copies:
        c.wait()

    lane = lax.iota(jnp.int32, _L)
    acc = jnp.zeros((_L,), jnp.float32)
    for i in range(b_per_w):
        acc = acc + jnp.where(lane == cols[i] % _L, win_v[i, :], 0.0)
    acc_v[...] = acc
    pltpu.sync_copy(acc_v, out_ref.at[wid])


@functools.cache
def _make_gather(B, V):
    b_per_w = B // _NW
    mesh = plsc.VectorSubcoreMesh(
        core_axis_name="c", subcore_axis_name="s",
        num_cores=_NC, num_subcores=_NS,
    )
    return pl.kernel(
        functools.partial(_sc_gather, b_per_w=b_per_w),
        out_type=jax.ShapeDtypeStruct((_NW, _L), jnp.float32),
        mesh=mesh,
        scratch_types=[
            pltpu.VMEM((b_per_w,), jnp.int32),
            pltpu.VMEM((b_per_w, _L), jnp.float32),
            pltpu.VMEM((_L,), jnp.float32),
            pltpu.SemaphoreType.DMA,
        ],
    )


def _tc_body(x_ref, cor_ref, out_ref, acc_ref, *, n_cols, n_rows):
    k = pl.program_id(0)

    @pl.when(k == 0)
    def _init():
        acc_ref[0] = 0.0

    x = x_ref[...]
    ids = k * _W + lax.broadcasted_iota(jnp.int32, x.shape, 1)
    acc_ref[0] += jnp.sum(jnp.where(ids < n_cols, x, 0.0))

    @pl.when(k == pl.num_programs(0) - 1)
    def _fin():
        csum = jnp.sum(cor_ref[...])
        val = (acc_ref[0] - (n_cols + 1.0) * csum) / n_rows
        out_ref[...] = jnp.reshape(val + (2.0 * n_cols - 1.0) * J, (1, 1))


def kernel(outputs, labels):
    B, V = outputs.shape
    lab = labels.astype(jnp.int32)
    cor_parts = _make_gather(B, V)(outputs, lab)

    n_blocks = pl.cdiv(V, _W)
    out = pl.pallas_call(
        functools.partial(_tc_body, n_cols=V, n_rows=B),
        grid=(n_blocks,),
        in_specs=[
            pl.BlockSpec((B, _W), lambda k: (0, k)),
            pl.BlockSpec((_NW, _L), lambda k: (0, 0)),
        ],
        out_specs=pl.BlockSpec((1, 1), lambda k: (0, 0)),
        out_shape=jax.ShapeDtypeStruct((1, 1), jnp.float32),
        scratch_shapes=[pltpu.SMEM((1,), jnp.float32)],
    )(outputs, cor_parts)
    return out[0, 0]


# speedup vs baseline: 2.1357x; 1.0029x over previous
import functools

import jax
import jax.numpy as jnp
from jax import lax
from jax.experimental import pallas as pl
from jax.experimental.pallas import tpu as pltpu
from jax.experimental.pallas import tpu_sc as plsc

J = 0.1
_R = 32

_NC, _NS = 2, 16
_NW = _NC * _NS
_L = 16


def _sc_gather(x_ref, lab_ref, out_ref, lab_v, win_v, acc_v, sem, *, b_per_w):
    wid = lax.axis_index("s") * _NC + lax.axis_index("c")
    base = wid * b_per_w
    pltpu.sync_copy(lab_ref.at[pl.ds(base, b_per_w)], lab_v)

    chunks = [lab_v[pl.ds(c * _L, _L)] for c in range(b_per_w // _L)]
    cols = [chunks[i // _L][i % _L] for i in range(b_per_w)]

    copies = []
    for i in range(b_per_w):
        a = (cols[i] // _L) * _L
        copies.append(
            pltpu.async_copy(x_ref.at[base + i, pl.ds(a, _L)], win_v.at[i], sem)
        )
    for c in copies:
        c.wait()

    lane = lax.iota(jnp.int32, _L)
    acc = jnp.zeros((_L,), jnp.float32)
    for i in range(b_per_w):
        acc = acc + jnp.where(lane == cols[i] % _L, win_v[i, :], 0.0)
    acc_v[...] = acc
    pltpu.sync_copy(acc_v, out_ref.at[wid])


@functools.cache
def _make_gather(B, V):
    b_per_w = B // _NW
    mesh = plsc.VectorSubcoreMesh(
        core_axis_name="c", subcore_axis_name="s",
        num_cores=_NC, num_subcores=_NS,
    )
    return pl.kernel(
        functools.partial(_sc_gather, b_per_w=b_per_w),
        out_type=jax.ShapeDtypeStruct((_NW, _L), jnp.float32),
        mesh=mesh,
        scratch_types=[
            pltpu.VMEM((b_per_w,), jnp.int32),
            pltpu.VMEM((b_per_w, _L), jnp.float32),
            pltpu.VMEM((_L,), jnp.float32),
            pltpu.SemaphoreType.DMA,
        ],
    )


def _tc_body(x_ref, cor_ref, out_ref, acc_ref, *, n_cols, n_rows):
    k = pl.program_id(0)

    @pl.when(k == 0)
    def _init():
        acc_ref[0] = 0.0

    acc_ref[0] += jnp.sum(x_ref[...])

    @pl.when(k == pl.num_programs(0) - 1)
    def _fin():
        csum = jnp.sum(cor_ref[...])
        val = (acc_ref[0] - (n_cols + 1.0) * csum) / n_rows
        out_ref[...] = jnp.reshape(val + (2.0 * n_cols - 1.0) * J, (1, 1))


def kernel(outputs, labels):
    B, V = outputs.shape
    lab = labels.astype(jnp.int32)
    cor_parts = _make_gather(B, V)(outputs, lab)

    out = pl.pallas_call(
        functools.partial(_tc_body, n_cols=V, n_rows=B),
        grid=(B // _R,),
        in_specs=[
            pl.BlockSpec((_R, V), lambda k: (k, 0)),
            pl.BlockSpec((_NW, _L), lambda k: (0, 0)),
        ],
        out_specs=pl.BlockSpec((1, 1), lambda k: (0, 0)),
        out_shape=jax.ShapeDtypeStruct((1, 1), jnp.float32),
        scratch_shapes=[pltpu.SMEM((1,), jnp.float32)],
    )(outputs, cor_parts)
    return out[0, 0]
